# baseline (speedup 1.0000x reference)
import functools

import jax
import jax.numpy as jnp
from jax import lax
from jax.experimental import pallas as pl
from jax.experimental.pallas import tpu as pltpu
from jax.experimental.pallas import tpu_sc as plsc

_LANES = 16


@functools.lru_cache(maxsize=None)
def _build_sc_gather(B, S, D):
    info = plsc.get_sparse_core_info()
    NC, NS = info.num_cores, info.num_subcores
    NW = NC * NS
    NB = B // NW
    C = 128
    NSLOT = 2 + S
    n_io_tiles = NB // C
    n_noise_tiles = (NB * S) // C

    mesh = plsc.VectorSubcoreMesh(core_axis_name="c", subcore_axis_name="s")

    @functools.partial(
        pl.kernel,
        out_type=jax.ShapeDtypeStruct((B * NSLOT, D), jnp.float32),
        mesh=mesh,
        compiler_params=pltpu.CompilerParams(use_tc_tiling_on_sc=False),
        scratch_types=[
            pltpu.VMEM((1, C), jnp.int32),
            pltpu.VMEM((1, C), jnp.int32),
            pltpu.VMEM((C, D), jnp.float32),
            pltpu.SemaphoreType.DMA,
        ],
    )
    def sc_kernel(iw_hbm, ow_hbm, nwf_hbm, in_emb, out_emb, out_hbm,
                  idx_v, dest_v, rows_v, sem):
        wid = lax.axis_index("s") * NC + lax.axis_index("c")
        lane = lax.iota(jnp.int32, _LANES)

        def do_tile(src_idx_hbm, src_base, table, dest_of_i):
            pltpu.sync_copy(src_idx_hbm.at[pl.ds(src_base, C)], idx_v.at[0])
            pltpu.async_copy(table.at[idx_v.at[0]], rows_v, sem).wait()
            for v in range(C // _LANES):
                i = lane + (v * _LANES)
                dest_v[0, pl.ds(v * _LANES, _LANES)] = dest_of_i(i)
            pltpu.async_copy(rows_v, out_hbm.at[dest_v.at[0]], sem).wait()

        def in_body(t, carry):
            gb = wid * NB + t * C
            do_tile(iw_hbm, gb, in_emb, lambda i: (gb + i) * NSLOT)
            return carry

        def out_body(t, carry):
            gb = wid * NB + t * C
            do_tile(ow_hbm, gb, out_emb, lambda i: (gb + i) * NSLOT + 1)
            return carry

        def noise_body(t, carry):
            n0 = wid * NB * S + t * C

            def dest_of_i(i):
                n = n0 + i
                q = (n.astype(jnp.float32) * (1.0 / S)).astype(jnp.int32)
                r = n - q * S
                q = q + jnp.where(r >= S, 1, 0) - jnp.where(r < 0, 1, 0)
                return n + 2 + 2 * q

            do_tile(nwf_hbm, n0, out_emb, dest_of_i)
            return carry

        lax.fori_loop(0, n_io_tiles, in_body, 0)
        lax.fori_loop(0, n_io_tiles, out_body, 0)
        lax.fori_loop(0, n_noise_tiles, noise_body, 0)

    return sc_kernel


def kernel(input_words, output_words, noise_words, in_embed, out_embed):
    B, S = noise_words.shape
    D = in_embed.shape[1]
    sc = _build_sc_gather(B, S, D)
    out_flat = sc(input_words, output_words, noise_words.reshape(B * S),
                  in_embed, out_embed)
    return out_flat.reshape(B, 2 + S, D)


# speedup vs baseline: 1.0860x; 1.0860x over previous
import functools

import jax
import jax.numpy as jnp
from jax import lax
from jax.experimental import pallas as pl
from jax.experimental.pallas import tpu as pltpu
from jax.experimental.pallas import tpu_sc as plsc

_LANES = 16


@functools.lru_cache(maxsize=None)
def _build_sc_gather(B, S, D):
    info = plsc.get_sparse_core_info()
    NC, NS = info.num_cores, info.num_subcores
    NW = NC * NS
    NB = B // NW
    C = 128
    K = 4
    NSLOT = 2 + S
    T_IO = NB // C
    T_NZ = (NB * S) // C
    NV = C // _LANES

    mesh = plsc.VectorSubcoreMesh(core_axis_name="c", subcore_axis_name="s")

    @functools.partial(
        pl.kernel,
        out_type=jax.ShapeDtypeStruct((B * NSLOT, D), jnp.float32),
        mesh=mesh,
        compiler_params=pltpu.CompilerParams(use_tc_tiling_on_sc=False),
        scratch_types=[
            pltpu.VMEM((NB,), jnp.int32),
            pltpu.VMEM((NB,), jnp.int32),
            pltpu.VMEM((NB * S,), jnp.int32),
            pltpu.VMEM((K, C), jnp.int32),
            pltpu.VMEM((K, C, D), jnp.float32),
            pltpu.SemaphoreType.DMA,
            pltpu.SemaphoreType.DMA,
        ],
    )
    def sc_kernel(iw_hbm, ow_hbm, nwf_hbm, in_emb, out_emb, out_hbm,
                  idx_in, idx_out, idx_nz, dest_v, rows_v, sem_g, sem_s):
        wid = lax.axis_index("s") * NC + lax.axis_index("c")
        lane = lax.iota(jnp.int32, _LANES)

        pltpu.sync_copy(iw_hbm.at[pl.ds(wid * NB, NB)], idx_in)
        pltpu.sync_copy(ow_hbm.at[pl.ds(wid * NB, NB)], idx_out)
        pltpu.sync_copy(nwf_hbm.at[pl.ds(wid * NB * S, NB * S)], idx_nz)

        def run_phase(T, idx_vmem, table, dest_of):
            def issue(t, slot):
                pltpu.async_copy(table.at[idx_vmem.at[pl.ds(t * C, C)]],
                                 rows_v.at[slot], sem_g)
                for v in range(NV):
                    dest_v[slot, pl.ds(v * _LANES, _LANES)] = \
                        dest_of(t, lane + v * _LANES)

            def wait_gather(slot):
                pltpu.make_async_copy(table.at[idx_vmem.at[pl.ds(0, C)]],
                                      rows_v.at[slot], sem_g).wait()

            def wait_scatter():
                pltpu.make_async_copy(rows_v.at[0],
                                      out_hbm.at[dest_v.at[0]], sem_s).wait()

            for k in range(K - 1):
                issue(k, k)

            def outer(to, carry):
                for k in range(K):
                    t = to * K + k
                    wait_gather(k)
                    if k == 0:
                        pl.when(to > 0)(wait_scatter)
                    else:
                        wait_scatter()
                    pltpu.async_copy(rows_v.at[k],
                                     out_hbm.at[dest_v.at[k]], sem_s)
                    nxt = t + K - 1

                    def issue_nxt(nxt=nxt, slot=(k - 1) % K):
                        issue(nxt, slot)

                    pl.when(nxt < T)(issue_nxt)
                return carry

            lax.fori_loop(0, T // K, outer, 0)
            wait_scatter()

        gb0 = wid * NB

        run_phase(T_IO, idx_in, in_emb,
                  lambda t, i: (gb0 + t * C + i) * NSLOT)
        run_phase(T_IO, idx_out, out_emb,
                  lambda t, i: (gb0 + t * C + i) * NSLOT + 1)

        def dest_noise(t, i):
            n = gb0 * S + t * C + i
            q = (n.astype(jnp.float32) * (1.0 / S)).astype(jnp.int32)
            r = n - q * S
            q = q + jnp.where(r >= S, 1, 0) - jnp.where(r < 0, 1, 0)
            return n + 2 + 2 * q

        run_phase(T_NZ, idx_nz, out_emb, dest_noise)

    return sc_kernel


def kernel(input_words, output_words, noise_words, in_embed, out_embed):
    B, S = noise_words.shape
    D = in_embed.shape[1]
    sc = _build_sc_gather(B, S, D)
    out_flat = sc(input_words, output_words, noise_words.reshape(B * S),
                  in_embed, out_embed)
    return out_flat.reshape(B, 2 + S, D)
